# baseline (speedup 1.0000x reference)
import functools

import jax
import jax.numpy as jnp
from jax import lax
from jax.experimental import pallas as pl
from jax.experimental.pallas import tpu as pltpu
from jax.experimental.pallas import tpu_sc as plsc

NUM_CODES = 100000
EMB = 32
BATCH = 16384
NT = 4
NC = 2
NS = 16
NW = NC * NS
B_PER_W = BATCH // NW
CH = 128
NCHUNK = B_PER_W // CH


@functools.cache
def _build():
    mesh = plsc.VectorSubcoreMesh(core_axis_name="c", subcore_axis_name="s")

    scratch = (
        [pltpu.VMEM((CH,), jnp.int32) for _ in range(NCHUNK)]
        + [pltpu.VMEM((CH, EMB), jnp.float32) for _ in range(NT * NCHUNK)]
        + [pltpu.SemaphoreType.DMA, pltpu.SemaphoreType.DMA]
    )

    @functools.partial(
        pl.kernel,
        mesh=mesh,
        out_type=jax.ShapeDtypeStruct((BATCH, NT * EMB), jnp.float32),
        compiler_params=pltpu.CompilerParams(use_tc_tiling_on_sc=False),
        scratch_types=scratch,
    )
    def sc_gather(ids_hbm, w0, w1, w2, w3, out_hbm, *scratch_refs):
        idx_v = scratch_refs[:NCHUNK]
        rows_v = scratch_refs[NCHUNK:NCHUNK + NT * NCHUNK]
        gsem, osem = scratch_refs[-2:]
        tables = [w0, w1, w2, w3]
        wid = lax.axis_index("s") * NC + lax.axis_index("c")
        base = wid * B_PER_W

        for j in range(NCHUNK):
            pltpu.sync_copy(ids_hbm.at[pl.ds(base + j * CH, CH)], idx_v[j])

        gathers = []
        for t in range(NT):
            for j in range(NCHUNK):
                gathers.append(
                    pltpu.async_copy(
                        tables[t].at[idx_v[j]], rows_v[t * NCHUNK + j], gsem
                    )
                )
        for g in gathers:
            g.wait()

        writes = []
        for t in range(NT):
            for j in range(NCHUNK):
                writes.append(
                    pltpu.async_copy(
                        rows_v[t * NCHUNK + j],
                        out_hbm.at[pl.ds(base + j * CH, CH),
                                   pl.ds(t * EMB, EMB)],
                        osem,
                    )
                )
        for w in writes:
            w.wait()

    return sc_gather


def kernel(code_ids, W0, W1, W2, W3):
    ids = code_ids.astype(jnp.int32)
    return _build()(ids, W0, W1, W2, W3)


# speedup vs baseline: 1.1462x; 1.1462x over previous
import functools

import jax
import jax.numpy as jnp
from jax import lax
from jax.experimental import pallas as pl
from jax.experimental.pallas import tpu as pltpu
from jax.experimental.pallas import tpu_sc as plsc

NUM_CODES = 100000
EMB = 32
BATCH = 16384
NT = 4
NC = 2
NS = 16
NW = NC * NS
B_PER_W = BATCH // NW
CH = 128
NCHUNK = B_PER_W // CH
D = NT * EMB


@functools.cache
def _build():
    mesh = plsc.VectorSubcoreMesh(core_axis_name="c", subcore_axis_name="s")

    scratch = (
        [pltpu.VMEM((CH,), jnp.int32) for _ in range(NCHUNK)]
        + [pltpu.VMEM((CH, D), jnp.float32) for _ in range(NCHUNK)]
        + [pltpu.SemaphoreType.DMA, pltpu.SemaphoreType.DMA]
    )

    @functools.partial(
        pl.kernel,
        mesh=mesh,
        out_type=jax.ShapeDtypeStruct((BATCH, D), jnp.float32),
        scratch_types=scratch,
    )
    def sc_gather(ids_hbm, wcat_hbm, out_hbm, *scratch_refs):
        idx_v = scratch_refs[:NCHUNK]
        rows_v = scratch_refs[NCHUNK:2 * NCHUNK]
        gsem, osem = scratch_refs[-2:]
        wid = lax.axis_index("s") * NC + lax.axis_index("c")
        base = wid * B_PER_W

        for j in range(NCHUNK):
            pltpu.sync_copy(ids_hbm.at[pl.ds(base + j * CH, CH)], idx_v[j])

        gathers = [
            pltpu.async_copy(wcat_hbm.at[idx_v[j]], rows_v[j], gsem)
            for j in range(NCHUNK)
        ]
        for g in gathers:
            g.wait()

        writes = [
            pltpu.async_copy(rows_v[j], out_hbm.at[pl.ds(base + j * CH, CH)],
                             osem)
            for j in range(NCHUNK)
        ]
        for w in writes:
            w.wait()

    return sc_gather


def kernel(code_ids, W0, W1, W2, W3):
    ids = code_ids.astype(jnp.int32)
    wcat = jnp.concatenate([W0, W1, W2, W3], axis=1)
    return _build()(ids, wcat)
